# Initial kernel scaffold; baseline (speedup 1.0000x reference)
import jax
import jax.numpy as jnp
from jax.experimental import pallas as pl


def kernel(cc_signals, h_prev, prev_messages, eff_prim, eff_key, eff_decay, conn_indices, branch_w, group_w):
    raise NotImplementedError("write your pallas kernel here")


# speedup vs baseline: 1.1763x; 1.1763x over previous
import functools

import jax
import jax.numpy as jnp
from jax import lax
from jax.experimental import pallas as pl
from jax.experimental.pallas import tpu as pltpu
from jax.experimental.pallas import tpu_sc as plsc

NB, BSZ, NG, BPG = 8, 4, 4, 2
L = 16
G = 4
NC, NS = 2, 16


def _sigmoid(x):
    return 1.0 / (1.0 + jnp.exp(-x))


def _tanh(x):
    return 1.0 - 2.0 / (jnp.exp(2.0 * x) + 1.0)


def _step_kernel_body(N, C, D, K,
                      msg_src, h_in, key_f, prim_f, dec_f, idx_f, bw_f, gw_f, cc_f,
                      h_out, msg_out,
                      idx_v, msgs_v, bw_v, gw_v, key_v, prim_v, h_v, dec_v, cc_v,
                      bb_v, hn_v, mn_v, sem):
    nch = D // L
    n_chunks = N // G
    ch_per = (n_chunks + NS - 1) // NS
    bs = lax.axis_index("c")
    sid = lax.axis_index("s")

    pltpu.sync_copy(cc_f.at[pl.ds(bs * C, C)], cc_v)

    def chunk_iter(j, _):
        chunk = sid + NS * j

        @pl.when(chunk < n_chunks)
        def _():
            n0 = chunk * G
            base = bs * N + n0
            pltpu.sync_copy(idx_f.at[pl.ds(base * K, G * K)], idx_v)
            gather = pltpu.async_copy(msg_src.at[idx_v], msgs_v, sem)
            pltpu.sync_copy(bw_f.at[pl.ds(n0, G)], bw_v)
            pltpu.sync_copy(gw_f.at[pl.ds(n0, G)], gw_v)
            pltpu.sync_copy(key_f.at[pl.ds(base, G)], key_v)
            pltpu.sync_copy(prim_f.at[pl.ds(base, G)], prim_v)
            pltpu.sync_copy(h_in.at[pl.ds(base, G)], h_v)
            pltpu.sync_copy(dec_f.at[pl.ds(base, G)], dec_v)
            gather.wait()

            def node_iter(i, _):
                node = n0 + i
                kv = [key_v[i, pl.ds(c * L, L)] for c in range(nch)]

                def b_iter(b, _):
                    bacc = [jnp.zeros((L,), jnp.float32) for _ in range(nch)]
                    for s in range(BSZ):
                        row = i * K + b * BSZ + s
                        m = [msgs_v[row, pl.ds(c * L, L)] for c in range(nch)]
                        part = m[0] * kv[0]
                        for c in range(1, nch):
                            part = part + m[c] * kv[c]
                        sim = jnp.sum(part)
                        rt = _sigmoid(jnp.broadcast_to(sim, (L,)))
                        for c in range(nch):
                            w = bw_v[i, b * BSZ + s, pl.ds(c * L, L)]
                            bacc[c] = bacc[c] + rt * (m[c] * w)
                    for c in range(nch):
                        bb_v[b, pl.ds(c * L, L)] = _tanh(bacc[c])
                    return 0

                lax.fori_loop(0, NB, b_iter, 0)

                recv = [jnp.zeros((L,), jnp.float32) for _ in range(nch)]
                for g in range(NG):
                    for c in range(nch):
                        sl = pl.ds(c * L, L)
                        ga = (bb_v[BPG * g, sl] * gw_v[i, BPG * g, sl]
                              + bb_v[BPG * g + 1, sl] * gw_v[i, BPG * g + 1, sl])
                        recv[c] = recv[c] + _tanh(ga)

                in_cc = node < C
                ccn = jnp.where(in_cc, node, 0)
                inv_ng = 1.0 / NG
                for c in range(nch):
                    sl = pl.ds(c * L, L)
                    r = recv[c] * inv_ng
                    r = r + jnp.where(in_cc, cc_v[ccn, sl], 0.0)
                    dv = dec_v[i]
                    hn = dv * h_v[i, sl] + (1.0 - dv) * r
                    hn_v[i, sl] = hn
                    mn_v[i, sl] = _tanh(hn * prim_v[i, sl])
                return 0

            lax.fori_loop(0, G, node_iter, 0)
            pltpu.sync_copy(hn_v, h_out.at[pl.ds(base, G)])
            pltpu.sync_copy(mn_v, msg_out.at[pl.ds(base, G)])

        return 0

    lax.fori_loop(0, ch_per, chunk_iter, 0)


@functools.partial(jax.jit, static_argnums=(9, 10, 11, 12))
def _step(msg_f, h_f, key_f, prim_f, dec_f, idx_f, bw_f, gw_f, cc_f, N, C, D, K):
    BSN = msg_f.shape[0]
    mesh = plsc.VectorSubcoreMesh(core_axis_name="c", subcore_axis_name="s",
                                  num_cores=NC, num_subcores=NS)
    body = functools.partial(_step_kernel_body, N, C, D, K)
    return pl.kernel(
        body,
        out_type=(
            jax.ShapeDtypeStruct((BSN, D), jnp.float32),
            jax.ShapeDtypeStruct((BSN, D), jnp.float32),
        ),
        mesh=mesh,
        compiler_params=pltpu.CompilerParams(needs_layout_passes=False),
        scratch_types=[
            pltpu.VMEM((G * K,), jnp.int32),
            pltpu.VMEM((G * K, D), jnp.float32),
            pltpu.VMEM((G, NB * BSZ, D), jnp.float32),
            pltpu.VMEM((G, NG * BPG, D), jnp.float32),
            pltpu.VMEM((G, D), jnp.float32),
            pltpu.VMEM((G, D), jnp.float32),
            pltpu.VMEM((G, D), jnp.float32),
            pltpu.VMEM((G, L), jnp.float32),
            pltpu.VMEM((16, D), jnp.float32),
            pltpu.VMEM((NB, D), jnp.float32),
            pltpu.VMEM((G, D), jnp.float32),
            pltpu.VMEM((G, D), jnp.float32),
            pltpu.SemaphoreType.DMA,
        ],
    )(msg_f, h_f, key_f, prim_f, dec_f, idx_f, bw_f, gw_f, cc_f)


def kernel(cc_signals, h_prev, prev_messages, eff_prim, eff_key, eff_decay,
           conn_indices, branch_w, group_w):
    BS, T, C, D = cc_signals.shape
    N, K = conn_indices.shape

    conn = conn_indices.astype(jnp.int32)
    idx_f = (conn[None] + (jnp.arange(BS, dtype=jnp.int32) * N)[:, None, None])
    idx_f = idx_f.reshape(BS * N * K)
    dec_f = jnp.broadcast_to(eff_decay[..., None], (BS, N, L)).reshape(BS * N, L)
    h_f = h_prev.reshape(BS * N, D)
    msg_f = prev_messages.reshape(BS * N, D)
    key_f = eff_key.reshape(BS * N, D)
    prim_f = eff_prim.reshape(BS * N, D)
    bw_f = branch_w.reshape(N, NB * BSZ, D)
    gw_f = group_w.reshape(N, NG * BPG, D)

    outs = []
    h, m = h_f, msg_f
    for t in range(T):
        cc_f = cc_signals[:, t].reshape(BS * C, D)
        h, m = _step(m, h, key_f, prim_f, dec_f, idx_f, bw_f, gw_f, cc_f,
                     N, C, D, K)
        outs.append(m.reshape(BS, N, D)[:, :C])

    output = jnp.stack(outs, axis=1)
    return output, h.reshape(BS, N, D)


# speedup vs baseline: 1.2474x; 1.0604x over previous
import functools

import jax
import jax.numpy as jnp
from jax import lax
from jax.experimental import pallas as pl
from jax.experimental.pallas import tpu as pltpu
from jax.experimental.pallas import tpu_sc as plsc

NB, BSZ, NG, BPG = 8, 4, 4, 2
L = 16
G = 4
NC, NS = 2, 16
NPS = 628
CH_MAX = NPS // G
CH_PAD = 160


def _sigmoid(x):
    return 1.0 / (1.0 + jnp.exp(-x))


def _tanh(x):
    return 1.0 - 2.0 / (jnp.exp(2.0 * x) + 1.0)


def _step_kernel_body(N, C, D, K,
                      msg_src, h_in, key_f, prim_f, dec_f, idx_f, bw_f, gw_f, cc_f,
                      h_out, msg_out,
                      idx_all, msgs_v, bw_v, gw_v, key_v, prim_v, h_v, dec_v,
                      cc_v, hn_v, mn_v, sem_g, sem_l, sem_o, sem_i):
    nch = D // L
    bs = lax.axis_index("c")
    sid = lax.axis_index("s")
    n_start = sid * NPS
    nodes_here = jnp.minimum(NPS, N - n_start)
    ch_count = nodes_here // G

    row0 = (bs * NS + sid) * CH_PAD
    pltpu.async_copy(idx_f.at[pl.ds(row0, CH_PAD)], idx_all, sem_i)
    pltpu.async_copy(cc_f.at[pl.ds(bs * C, C)], cc_v, sem_i)
    pltpu.make_async_copy(idx_f.at[pl.ds(0, CH_PAD)], idx_all, sem_i).wait()
    pltpu.make_async_copy(cc_f.at[pl.ds(0, C)], cc_v, sem_i).wait()

    idx3 = idx_all

    def issue(j, p):
        @pl.when(j < ch_count)
        def _():
            n0 = n_start + j * G
            base = bs * N + n0
            pltpu.async_copy(msg_src.at[idx3.at[j]], msgs_v.at[p], sem_g.at[p])
            pltpu.async_copy(bw_f.at[pl.ds(n0, G)], bw_v.at[p], sem_l.at[p])
            pltpu.async_copy(gw_f.at[pl.ds(n0, G)], gw_v.at[p], sem_l.at[p])
            pltpu.async_copy(key_f.at[pl.ds(base, G)], key_v.at[p], sem_l.at[p])
            pltpu.async_copy(prim_f.at[pl.ds(base, G)], prim_v.at[p], sem_l.at[p])
            pltpu.async_copy(h_in.at[pl.ds(base, G)], h_v.at[p], sem_l.at[p])
            pltpu.async_copy(dec_f.at[pl.ds(base, G)], dec_v.at[p], sem_l.at[p])

    def wait_in(j, p):
        pltpu.make_async_copy(msg_src.at[idx3.at[j]], msgs_v.at[p],
                              sem_g.at[p]).wait()
        n0 = n_start + j * G
        base = bs * N + n0
        pltpu.make_async_copy(bw_f.at[pl.ds(n0, G)], bw_v.at[p], sem_l.at[p]).wait()
        pltpu.make_async_copy(gw_f.at[pl.ds(n0, G)], gw_v.at[p], sem_l.at[p]).wait()
        pltpu.make_async_copy(key_f.at[pl.ds(base, G)], key_v.at[p], sem_l.at[p]).wait()
        pltpu.make_async_copy(prim_f.at[pl.ds(base, G)], prim_v.at[p], sem_l.at[p]).wait()
        pltpu.make_async_copy(h_in.at[pl.ds(base, G)], h_v.at[p], sem_l.at[p]).wait()
        pltpu.make_async_copy(dec_f.at[pl.ds(base, G)], dec_v.at[p], sem_l.at[p]).wait()

    def wait_out(j, p):
        n0 = n_start + j * G
        base = bs * N + n0
        pltpu.make_async_copy(hn_v.at[p], h_out.at[pl.ds(base, G)],
                              sem_o.at[p]).wait()
        pltpu.make_async_copy(mn_v.at[p], msg_out.at[pl.ds(base, G)],
                              sem_o.at[p]).wait()

    issue(0, 0)

    def chunk_iter(j, _):
        p = j % 2
        issue(j + 1, (j + 1) % 2)

        @pl.when(j < ch_count)
        def _():
            wait_in(j, p)
            @pl.when(j >= 2)
            def _():
                wait_out(j - 2, p)

            def node_iter(i, _):
                node = n_start + j * G + i
                kv = [key_v[p, i, pl.ds(c * L, L)] for c in range(nch)]

                recv = [jnp.zeros((L,), jnp.float32) for _ in range(nch)]
                for g in range(NG):
                    gacc = [jnp.zeros((L,), jnp.float32) for _ in range(nch)]
                    for bp in range(BPG):
                        b = g * BPG + bp
                        bacc = [jnp.zeros((L,), jnp.float32) for _ in range(nch)]
                        for s in range(BSZ):
                            kk = b * BSZ + s
                            row = i * K + kk
                            m = [msgs_v[p, row, pl.ds(c * L, L)]
                                 for c in range(nch)]
                            e = [m[c] * kv[c] for c in range(nch)]
                            e0 = (e[0] + e[1]) + (e[2] + e[3])
                            e1 = (e[4] + e[5]) + (e[6] + e[7])
                            sim = jnp.sum(e0 + e1)
                            rt = _sigmoid(jnp.broadcast_to(sim, (L,)))
                            for c in range(nch):
                                w = bw_v[p, i, kk, pl.ds(c * L, L)]
                                bacc[c] = bacc[c] + rt * (m[c] * w)
                        for c in range(nch):
                            gw = gw_v[p, i, b, pl.ds(c * L, L)]
                            gacc[c] = gacc[c] + _tanh(bacc[c]) * gw
                    for c in range(nch):
                        recv[c] = recv[c] + _tanh(gacc[c])

                inv_ng = 1.0 / NG

                def add_cc(rv):
                    ccn = jnp.minimum(node, C - 1)
                    return [rv[c] + cc_v[ccn, pl.ds(c * L, L)]
                            for c in range(nch)]

                recv = lax.cond(node < C, add_cc, lambda rv: list(rv),
                                [r * inv_ng for r in recv])

                for c in range(nch):
                    sl = pl.ds(c * L, L)
                    dv = dec_v[p, i]
                    hn = dv * h_v[p, i, sl] + (1.0 - dv) * recv[c]
                    hn_v[p, i, sl] = hn
                    mn_v[p, i, sl] = _tanh(hn * prim_v[p, i, sl])
                return 0

            lax.fori_loop(0, G, node_iter, 0)
            n0 = n_start + j * G
            base = bs * N + n0
            pltpu.async_copy(hn_v.at[p], h_out.at[pl.ds(base, G)], sem_o.at[p])
            pltpu.async_copy(mn_v.at[p], msg_out.at[pl.ds(base, G)], sem_o.at[p])

        return 0

    lax.fori_loop(0, CH_MAX, chunk_iter, 0)

    @pl.when(ch_count >= 2)
    def _():
        wait_out(ch_count - 2, (ch_count - 2) % 2)

    @pl.when(ch_count >= 1)
    def _():
        wait_out(ch_count - 1, (ch_count - 1) % 2)


@functools.partial(jax.jit, static_argnums=(9, 10, 11, 12))
def _step(msg_f, h_f, key_f, prim_f, dec_f, idx_f, bw_f, gw_f, cc_f, N, C, D, K):
    BSN = msg_f.shape[0]
    mesh = plsc.VectorSubcoreMesh(core_axis_name="c", subcore_axis_name="s",
                                  num_cores=NC, num_subcores=NS)
    body = functools.partial(_step_kernel_body, N, C, D, K)
    return pl.kernel(
        body,
        out_type=(
            jax.ShapeDtypeStruct((BSN, D), jnp.float32),
            jax.ShapeDtypeStruct((BSN, D), jnp.float32),
        ),
        mesh=mesh,
        compiler_params=pltpu.CompilerParams(needs_layout_passes=False),
        scratch_types=[
            pltpu.VMEM((CH_PAD, G * K), jnp.int32),
            pltpu.VMEM((2, G * K, D), jnp.float32),
            pltpu.VMEM((2, G, NB * BSZ, D), jnp.float32),
            pltpu.VMEM((2, G, NG * BPG, D), jnp.float32),
            pltpu.VMEM((2, G, D), jnp.float32),
            pltpu.VMEM((2, G, D), jnp.float32),
            pltpu.VMEM((2, G, D), jnp.float32),
            pltpu.VMEM((2, G, L), jnp.float32),
            pltpu.VMEM((16, D), jnp.float32),
            pltpu.VMEM((2, G, D), jnp.float32),
            pltpu.VMEM((2, G, D), jnp.float32),
            pltpu.SemaphoreType.DMA((2,)),
            pltpu.SemaphoreType.DMA((2,)),
            pltpu.SemaphoreType.DMA((2,)),
            pltpu.SemaphoreType.DMA,
        ],
    )(msg_f, h_f, key_f, prim_f, dec_f, idx_f, bw_f, gw_f, cc_f)


def kernel(cc_signals, h_prev, prev_messages, eff_prim, eff_key, eff_decay,
           conn_indices, branch_w, group_w):
    BS, T, C, D = cc_signals.shape
    N, K = conn_indices.shape
    n_pad = NS * NPS

    conn = conn_indices.astype(jnp.int32)
    conn = jnp.pad(conn, ((0, n_pad - N), (0, 0)))
    idx_f = (conn[None] + (jnp.arange(BS, dtype=jnp.int32) * N)[:, None, None])
    idx_f = idx_f.reshape(BS, NS, CH_MAX, G * K)
    idx_f = jnp.pad(idx_f, ((0, 0), (0, 0), (0, CH_PAD - CH_MAX), (0, 0)))
    idx_f = idx_f.reshape(BS * NS * CH_PAD, G * K)
    dec_f = jnp.broadcast_to(eff_decay[..., None], (BS, N, L)).reshape(BS * N, L)
    h_f = h_prev.reshape(BS * N, D)
    msg_f = prev_messages.reshape(BS * N, D)
    key_f = eff_key.reshape(BS * N, D)
    prim_f = eff_prim.reshape(BS * N, D)
    bw_f = branch_w.reshape(N, NB * BSZ, D)
    gw_f = group_w.reshape(N, NG * BPG, D)

    outs = []
    h, m = h_f, msg_f
    for t in range(T):
        cc_f = cc_signals[:, t].reshape(BS * C, D)
        h, m = _step(m, h, key_f, prim_f, dec_f, idx_f, bw_f, gw_f, cc_f,
                     N, C, D, K)
        outs.append(m.reshape(BS, N, D)[:, :C])

    output = jnp.stack(outs, axis=1)
    return output, h.reshape(BS, N, D)
